# baseline (speedup 1.0000x reference)
import functools

import jax
import jax.numpy as jnp
from jax import lax
from jax.experimental import pallas as pl
from jax.experimental.pallas import tpu as pltpu
from jax.experimental.pallas import tpu_sc as plsc

N_NODES = 100000
N_EDGES = 1600000
NPAD = 102400
NC = 2
NS = 16
NW = NC * NS
NBLK = 8

_sc_params = pltpu.CompilerParams(use_tc_tiling_on_sc=False)


EPW = N_EDGES // NW
KA = 2000
NCHUNK_A = EPW // KA

_sc_mesh = plsc.VectorSubcoreMesh(core_axis_name="c", subcore_axis_name="s",
                                  num_cores=NC, num_subcores=NS)


@functools.partial(
    pl.kernel,
    out_type=jax.ShapeDtypeStruct((N_EDGES, 16), jnp.float32),
    mesh=_sc_mesh,
    scratch_types=[
        pltpu.VMEM((KA,), jnp.int32),
        pltpu.VMEM((KA,), jnp.int32),
        pltpu.VMEM((KA, 16), jnp.float32),
        pltpu.VMEM((KA, 16), jnp.float32),
        pltpu.SemaphoreType.DMA,
        pltpu.SemaphoreType.DMA,
        pltpu.SemaphoreType.DMA,
        pltpu.SemaphoreType.DMA,
        pltpu.SemaphoreType.DMA,
        pltpu.SemaphoreType.DMA,
    ],
    compiler_params=_sc_params,
)
def _phase_a(x_hbm, col_hbm,
             xg_hbm,
             col0, col1, xr0, xr1, sc0, sc1, sg0, sg1, sw0, sw1):
    c = lax.axis_index("c")
    s = lax.axis_index("s")
    wid = s * NC + c
    base_w = wid * EPW
    cbufs = (col0, col1)
    xbufs = (xr0, xr1)
    csems = (sc0, sc1)
    gsems = (sg0, sg1)
    wsems = (sw0, sw1)

    def icol(g, b):
        pltpu.async_copy(col_hbm.at[pl.ds(base_w + g * KA, KA)],
                         cbufs[b], csems[b])

    def wcol(g, b):
        pltpu.make_async_copy(col_hbm.at[pl.ds(base_w + g * KA, KA)],
                              cbufs[b], csems[b]).wait()

    def gather(b):
        pltpu.async_copy(x_hbm.at[cbufs[b]], xbufs[b], gsems[b]).wait()

    def iwr(g, b):
        pltpu.async_copy(xbufs[b], xg_hbm.at[pl.ds(base_w + g * KA, KA), :],
                         wsems[b])

    def wwr(g, b):
        pltpu.make_async_copy(xbufs[b],
                              xg_hbm.at[pl.ds(base_w + g * KA, KA), :],
                              wsems[b]).wait()

    icol(0, 0)
    icol(1, 1)

    def body(it, carry):
        g = it * 2
        for b, gg in ((0, g), (1, g + 1)):
            wcol(gg, b)

            @pl.when(it > 0)
            def _(b=b, gg=gg):
                wwr(gg - 2, b)

            gather(b)
            iwr(gg, b)

            @pl.when(gg + 2 < NCHUNK_A)
            def _(b=b, gg=gg):
                icol(gg + 2, b)

        return carry

    lax.fori_loop(0, NCHUNK_A // 2, body, 0)
    g_last = NCHUNK_A - 1
    wcol(g_last, 0)
    wwr(g_last - 2, 0)
    gather(0)
    iwr(g_last, 0)
    wwr(g_last - 1, 1)
    wwr(g_last, 0)


BE = 6400
EH = N_EDGES // 2


def _phase_b_body(xg, ea_t, w1a, w1b, b1, w2, b2, h2):
    h1 = jnp.dot(xg[...], w1a[...], preferred_element_type=jnp.float32)
    h1 = h1 + lax.dot_general(ea_t[...], w1b[...],
                              dimension_numbers=(((0,), (0,)), ((), ())),
                              preferred_element_type=jnp.float32)
    h1 = jnp.maximum(h1 + b1[...], 0.0)
    h = jnp.dot(h1, w2[...], preferred_element_type=jnp.float32) + b2[...]
    h2[...] = jnp.maximum(h, 0.0)


def _edge_mlp(half, xg, ea_t, w1a, w1b, b1, w2, b2):
    grid = (EH // BE,)
    off = half * (EH // BE)
    return pl.pallas_call(
        _phase_b_body,
        grid=grid,
        in_specs=[
            pl.BlockSpec((BE, 16), lambda i: (i + off, 0)),
            pl.BlockSpec((39, BE), lambda i: (0, i + off)),
            pl.BlockSpec((16, 64), lambda i: (0, 0)),
            pl.BlockSpec((39, 64), lambda i: (0, 0)),
            pl.BlockSpec((1, 64), lambda i: (0, 0)),
            pl.BlockSpec((64, 128), lambda i: (0, 0)),
            pl.BlockSpec((1, 128), lambda i: (0, 0)),
        ],
        out_specs=pl.BlockSpec((BE, 128), lambda i: (i, 0)),
        out_shape=jax.ShapeDtypeStruct((EH, 128), jnp.float32),
    )(xg, ea_t, w1a, w1b, b1, w2, b2)


EPT_H = EH // NS
GC = 800
NCH_H = EPT_H // GC
GT = 400
EPT_K = N_EDGES // NS // NC
NCH_K = EPT_K // GC
NBLK_PER_CORE = NBLK // NC


def _make_phase_c(half, with_counts):
    out_type = [jax.ShapeDtypeStruct((NPAD, 128), jnp.float32)]
    if with_counts:
        out_type += [jax.ShapeDtypeStruct((NPAD, 16), jnp.float32)] * 2

    @functools.partial(
        pl.kernel,
        out_type=out_type,
        mesh=_sc_mesh,
        scratch_types=[
            pltpu.VMEM((GC,), jnp.int32),
            pltpu.VMEM((GC,), jnp.int32),
            pltpu.VMEM((GT,), jnp.int32),
            pltpu.VMEM((GC, 16), jnp.float32),
            pltpu.VMEM((GC, 16), jnp.float32),
            pltpu.VMEM_SHARED((NPAD, 16), jnp.float32),
            pltpu.SemaphoreType.DMA,
            pltpu.SemaphoreType.DMA,
            pltpu.SemaphoreType.DMA,
            pltpu.SemaphoreType.DMA,
            pltpu.SemaphoreType.DMA,
            pltpu.SemaphoreType.DMA,
        ],
        compiler_params=_sc_params,
    )
    def _phase_c(h2_hbm, row_hbm, zeros2_hbm, ones_hbm,
                 *refs):
        if with_counts:
            sums_hbm, k0, k1 = refs[0], refs[1], refs[2]
            scratch = refs[3:]
        else:
            sums_hbm = refs[0]
            scratch = refs[1:]
        (idx0, idx1, idxtk, rows0, rows1, acc,
         si0, si1, sr0, sr1, ss0, ss1) = scratch
        klist = (k0, k1) if with_counts else None
        ibufs = (idx0, idx1)
        rbufs = (rows0, rows1)
        isems = (si0, si1)
        rsems = (sr0, sr1)
        ssems = (ss0, ss1)
        c = lax.axis_index("c")
        s = lax.axis_index("s")

        for p in range(NBLK_PER_CORE):
            for cc in range(NC):
                blk = cc * NBLK_PER_CORE + p
                colbase = blk * 16

                @pl.when(c == cc)
                def _(colbase=colbase):
                    lbase = s * EPT_H
                    rbase = half * EH + s * EPT_H

                    def issue(g, b):
                        pltpu.async_copy(
                            row_hbm.at[pl.ds(rbase + g * GC, GC)],
                            ibufs[b], isems[b])
                        pltpu.async_copy(
                            h2_hbm.at[pl.ds(lbase + g * GC, GC),
                                      pl.ds(colbase, 16)],
                            rbufs[b], rsems[b])

                    def wait_load(g, b):
                        pltpu.make_async_copy(
                            row_hbm.at[pl.ds(rbase + g * GC, GC)],
                            ibufs[b], isems[b]).wait()
                        pltpu.make_async_copy(
                            h2_hbm.at[pl.ds(lbase + g * GC, GC),
                                      pl.ds(colbase, 16)],
                            rbufs[b], rsems[b]).wait()

                    def scatter(b):
                        pltpu.async_copy(rbufs[b], acc.at[ibufs[b]],
                                         ssems[b], add=True)
                        pltpu.make_async_copy(rbufs[b], acc.at[ibufs[b]],
                                              ssems[b]).wait()

                    pltpu.sync_copy(zeros2_hbm,
                                    acc.at[pl.ds(s * 6400, 6400), :])
                    plsc.subcore_barrier()
                    issue(0, 0)
                    issue(1, 1)

                    def body(it, carry):
                        g = it * 2
                        wait_load(g, 0)
                        scatter(0)

                        @pl.when(g + 2 < NCH_H)
                        def _():
                            issue(g + 2, 0)

                        wait_load(g + 1, 1)
                        scatter(1)

                        @pl.when(g + 3 < NCH_H)
                        def _():
                            issue(g + 3, 1)

                        return carry

                    lax.fori_loop(0, NCH_H // 2, body, 0)
                    pltpu.sync_copy(
                        row_hbm.at[pl.ds(rbase + NCH_H * GC, GT)], idxtk)
                    pltpu.sync_copy(
                        h2_hbm.at[pl.ds(lbase + NCH_H * GC, GT),
                                  pl.ds(colbase, 16)],
                        rows0.at[pl.ds(0, GT), :])
                    pltpu.sync_copy(rows0.at[pl.ds(0, GT), :],
                                    acc.at[idxtk], add=True)
                    plsc.subcore_barrier()
                    pltpu.sync_copy(acc.at[pl.ds(s * 6400, 6400), :],
                                    sums_hbm.at[pl.ds(s * 6400, 6400),
                                                pl.ds(colbase, 16)])
                    plsc.subcore_barrier()

        if not with_counts:
            return

        pltpu.sync_copy(ones_hbm, rows0)
        for cc in range(NC):

            @pl.when(c == cc)
            def _(cc=cc):
                ebase = cc * (N_EDGES // NC) + s * EPT_K

                def issue_k(g, b):
                    pltpu.async_copy(row_hbm.at[pl.ds(ebase + g * GC, GC)],
                                     ibufs[b], isems[b])

                def wait_scatter_k(g, b):
                    pltpu.make_async_copy(
                        row_hbm.at[pl.ds(ebase + g * GC, GC)],
                        ibufs[b], isems[b]).wait()
                    pltpu.sync_copy(rows0, acc.at[ibufs[b]], add=True)

                pltpu.sync_copy(zeros2_hbm, acc.at[pl.ds(s * 6400, 6400), :])
                plsc.subcore_barrier()
                issue_k(0, 0)
                issue_k(1, 1)

                def body(it, carry):
                    g = it * 2
                    wait_scatter_k(g, 0)

                    @pl.when(g + 2 < NCH_K)
                    def _():
                        issue_k(g + 2, 0)

                    wait_scatter_k(g + 1, 1)

                    @pl.when(g + 3 < NCH_K)
                    def _():
                        issue_k(g + 3, 1)

                    return carry

                lax.fori_loop(0, NCH_K // 2, body, 0)
                pltpu.sync_copy(row_hbm.at[pl.ds(ebase + NCH_K * GC, GT)],
                                idxtk)
                pltpu.sync_copy(rows0.at[pl.ds(0, GT), :], acc.at[idxtk],
                                add=True)
                plsc.subcore_barrier()
                pltpu.sync_copy(acc.at[pl.ds(s * 6400, 6400), :],
                                klist[cc].at[pl.ds(s * 6400, 6400), :])
                plsc.subcore_barrier()

    return _phase_c


_phase_c_a = _make_phase_c(0, False)
_phase_c_b = _make_phase_c(1, True)


BN = 2000


def _phase_d_body(x, c0, c1, sa, sb, w3a, w3b, b3, w4, b4, out):
    ctot = c0[:, 0:1] + c1[:, 0:1]
    inv = 1.0 / jnp.maximum(ctot, 1.0)
    mean = (sa[...] + sb[...]) * inv
    h = jnp.dot(x[...], w3a[...], preferred_element_type=jnp.float32)
    h = h + jnp.dot(mean, w3b[...], preferred_element_type=jnp.float32)
    h = jnp.maximum(h + b3[...], 0.0)
    o = jnp.dot(h, w4[...], preferred_element_type=jnp.float32) + b4[...]
    out[...] = jnp.maximum(o, 0.0)


def _node_mlp(x, c0, c1, sa, sb, w3a, w3b, b3, w4, b4):
    grid = (N_NODES // BN,)
    return pl.pallas_call(
        _phase_d_body,
        grid=grid,
        in_specs=[
            pl.BlockSpec((BN, 4), lambda i: (i, 0)),
            pl.BlockSpec((BN, 16), lambda i: (i, 0)),
            pl.BlockSpec((BN, 16), lambda i: (i, 0)),
            pl.BlockSpec((BN, 128), lambda i: (i, 0)),
            pl.BlockSpec((BN, 128), lambda i: (i, 0)),
            pl.BlockSpec((4, 256), lambda i: (0, 0)),
            pl.BlockSpec((128, 256), lambda i: (0, 0)),
            pl.BlockSpec((1, 256), lambda i: (0, 0)),
            pl.BlockSpec((256, 256), lambda i: (0, 0)),
            pl.BlockSpec((1, 256), lambda i: (0, 0)),
        ],
        out_specs=pl.BlockSpec((BN, 256), lambda i: (i, 0)),
        out_shape=jax.ShapeDtypeStruct((N_NODES, 256), jnp.float32),
    )(x, c0, c1, sa, sb, w3a, w3b, b3, w4, b4)


def kernel(x, edge_index, edge_attr, W1, b1, W2, b2, W3, b3, W4, b4):
    row = edge_index[0]
    col = edge_index[1]
    x_pad = jnp.pad(x, ((0, 0), (0, 12)))
    w1a_pad = jnp.pad(W1[:4], ((0, 12), (0, 0)))
    ea_t = edge_attr.T
    zeros2 = jnp.zeros((6400, 16), jnp.float32)
    ones_c = jnp.ones((GC, 16), jnp.float32)
    b1r = b1.reshape(1, 64)
    b2r = b2.reshape(1, 128)

    xg = _phase_a(x_pad, col)
    h2a = _edge_mlp(0, xg, ea_t, w1a_pad, W1[4:], b1r, W2, b2r)
    h2b = _edge_mlp(1, xg, ea_t, w1a_pad, W1[4:], b1r, W2, b2r)
    (sums_a,) = _phase_c_a(h2a, row, zeros2, ones_c)
    sums_b, cnt0, cnt1 = _phase_c_b(h2b, row, zeros2, ones_c)
    out = _node_mlp(x, cnt0, cnt1, sums_a, sums_b, W3[:4], W3[4:],
                    b3.reshape(1, 256), W4, b4.reshape(1, 256))
    return out


# speedup vs baseline: 1.0095x; 1.0095x over previous
import functools

import jax
import jax.numpy as jnp
from jax import lax
from jax.experimental import pallas as pl
from jax.experimental.pallas import tpu as pltpu
from jax.experimental.pallas import tpu_sc as plsc

N_NODES = 100000
N_EDGES = 1600000
NPAD = 102400
NC = 2
NS = 16
NW = NC * NS
NBLK = 8

_sc_params = pltpu.CompilerParams(use_tc_tiling_on_sc=False)


EPW = N_EDGES // NW
KA = 2000
NCHUNK_A = EPW // KA

_sc_mesh = plsc.VectorSubcoreMesh(core_axis_name="c", subcore_axis_name="s",
                                  num_cores=NC, num_subcores=NS)


@functools.partial(
    pl.kernel,
    out_type=jax.ShapeDtypeStruct((N_EDGES, 16), jnp.float32),
    mesh=_sc_mesh,
    scratch_types=[
        pltpu.VMEM((KA,), jnp.int32),
        pltpu.VMEM((KA,), jnp.int32),
        pltpu.VMEM((KA, 16), jnp.float32),
        pltpu.VMEM((KA, 16), jnp.float32),
        pltpu.SemaphoreType.DMA,
        pltpu.SemaphoreType.DMA,
        pltpu.SemaphoreType.DMA,
        pltpu.SemaphoreType.DMA,
        pltpu.SemaphoreType.DMA,
        pltpu.SemaphoreType.DMA,
    ],
    compiler_params=_sc_params,
)
def _phase_a(x_hbm, col_hbm,
             xg_hbm,
             col0, col1, xr0, xr1, sc0, sc1, sg0, sg1, sw0, sw1):
    c = lax.axis_index("c")
    s = lax.axis_index("s")
    wid = s * NC + c
    base_w = wid * EPW
    cbufs = (col0, col1)
    xbufs = (xr0, xr1)
    csems = (sc0, sc1)
    gsems = (sg0, sg1)
    wsems = (sw0, sw1)

    def icol(g, b):
        pltpu.async_copy(col_hbm.at[pl.ds(base_w + g * KA, KA)],
                         cbufs[b], csems[b])

    def wcol(g, b):
        pltpu.make_async_copy(col_hbm.at[pl.ds(base_w + g * KA, KA)],
                              cbufs[b], csems[b]).wait()

    def gather(b):
        pltpu.async_copy(x_hbm.at[cbufs[b]], xbufs[b], gsems[b]).wait()

    def iwr(g, b):
        pltpu.async_copy(xbufs[b], xg_hbm.at[pl.ds(base_w + g * KA, KA), :],
                         wsems[b])

    def wwr(g, b):
        pltpu.make_async_copy(xbufs[b],
                              xg_hbm.at[pl.ds(base_w + g * KA, KA), :],
                              wsems[b]).wait()

    icol(0, 0)
    icol(1, 1)

    def body(it, carry):
        g = it * 2
        for b, gg in ((0, g), (1, g + 1)):
            wcol(gg, b)

            @pl.when(it > 0)
            def _(b=b, gg=gg):
                wwr(gg - 2, b)

            gather(b)
            iwr(gg, b)

            @pl.when(gg + 2 < NCHUNK_A)
            def _(b=b, gg=gg):
                icol(gg + 2, b)

        return carry

    lax.fori_loop(0, NCHUNK_A // 2, body, 0)
    g_last = NCHUNK_A - 1
    wcol(g_last, 0)
    wwr(g_last - 2, 0)
    gather(0)
    iwr(g_last, 0)
    wwr(g_last - 1, 1)
    wwr(g_last, 0)


BE = 16000
EH = N_EDGES // 2


def _phase_b_body(xg, ea_t, w1a, w1b, b1, w2, b2, h2):
    h1 = jnp.dot(xg[...], w1a[...], preferred_element_type=jnp.float32)
    h1 = h1 + lax.dot_general(ea_t[...], w1b[...],
                              dimension_numbers=(((0,), (0,)), ((), ())),
                              preferred_element_type=jnp.float32)
    h1 = jnp.maximum(h1 + b1[...], 0.0)
    h = jnp.dot(h1, w2[...], preferred_element_type=jnp.float32) + b2[...]
    h2[...] = jnp.maximum(h, 0.0)


def _edge_mlp(half, xg, ea_t, w1a, w1b, b1, w2, b2):
    grid = (EH // BE,)
    off = half * (EH // BE)
    return pl.pallas_call(
        _phase_b_body,
        grid=grid,
        in_specs=[
            pl.BlockSpec((BE, 16), lambda i: (i + off, 0)),
            pl.BlockSpec((39, BE), lambda i: (0, i + off)),
            pl.BlockSpec((16, 64), lambda i: (0, 0)),
            pl.BlockSpec((39, 64), lambda i: (0, 0)),
            pl.BlockSpec((1, 64), lambda i: (0, 0)),
            pl.BlockSpec((64, 128), lambda i: (0, 0)),
            pl.BlockSpec((1, 128), lambda i: (0, 0)),
        ],
        out_specs=pl.BlockSpec((BE, 128), lambda i: (i, 0)),
        out_shape=jax.ShapeDtypeStruct((EH, 128), jnp.float32),
    )(xg, ea_t, w1a, w1b, b1, w2, b2)


EPT_H = EH // NS
GC = 800
NCH_H = EPT_H // GC
GT = 400
EPT_K = N_EDGES // NS // NC
NCH_K = EPT_K // GC
NBLK_PER_CORE = NBLK // NC


def _make_phase_c(half, with_counts):
    out_type = [jax.ShapeDtypeStruct((NPAD, 128), jnp.float32)]
    if with_counts:
        out_type += [jax.ShapeDtypeStruct((NPAD, 16), jnp.float32)] * 2

    @functools.partial(
        pl.kernel,
        out_type=out_type,
        mesh=_sc_mesh,
        scratch_types=[
            pltpu.VMEM((GC,), jnp.int32),
            pltpu.VMEM((GC,), jnp.int32),
            pltpu.VMEM((GT,), jnp.int32),
            pltpu.VMEM((GC, 16), jnp.float32),
            pltpu.VMEM((GC, 16), jnp.float32),
            pltpu.VMEM_SHARED((NPAD, 16), jnp.float32),
            pltpu.SemaphoreType.DMA,
            pltpu.SemaphoreType.DMA,
            pltpu.SemaphoreType.DMA,
            pltpu.SemaphoreType.DMA,
            pltpu.SemaphoreType.DMA,
            pltpu.SemaphoreType.DMA,
        ],
        compiler_params=_sc_params,
    )
    def _phase_c(h2_hbm, row_hbm, zeros2_hbm, ones_hbm,
                 *refs):
        if with_counts:
            sums_hbm, k0, k1 = refs[0], refs[1], refs[2]
            scratch = refs[3:]
        else:
            sums_hbm = refs[0]
            scratch = refs[1:]
        (idx0, idx1, idxtk, rows0, rows1, acc,
         si0, si1, sr0, sr1, ss0, ss1) = scratch
        klist = (k0, k1) if with_counts else None
        ibufs = (idx0, idx1)
        rbufs = (rows0, rows1)
        isems = (si0, si1)
        rsems = (sr0, sr1)
        ssems = (ss0, ss1)
        c = lax.axis_index("c")
        s = lax.axis_index("s")

        for p in range(NBLK_PER_CORE):
            for cc in range(NC):
                blk = cc * NBLK_PER_CORE + p
                colbase = blk * 16

                @pl.when(c == cc)
                def _(colbase=colbase):
                    lbase = s * EPT_H
                    rbase = half * EH + s * EPT_H

                    def issue(g, b):
                        pltpu.async_copy(
                            row_hbm.at[pl.ds(rbase + g * GC, GC)],
                            ibufs[b], isems[b])
                        pltpu.async_copy(
                            h2_hbm.at[pl.ds(lbase + g * GC, GC),
                                      pl.ds(colbase, 16)],
                            rbufs[b], rsems[b])

                    def wait_load(g, b):
                        pltpu.make_async_copy(
                            row_hbm.at[pl.ds(rbase + g * GC, GC)],
                            ibufs[b], isems[b]).wait()
                        pltpu.make_async_copy(
                            h2_hbm.at[pl.ds(lbase + g * GC, GC),
                                      pl.ds(colbase, 16)],
                            rbufs[b], rsems[b]).wait()

                    def scatter(b):
                        pltpu.async_copy(rbufs[b], acc.at[ibufs[b]],
                                         ssems[b], add=True)
                        pltpu.make_async_copy(rbufs[b], acc.at[ibufs[b]],
                                              ssems[b]).wait()

                    pltpu.sync_copy(zeros2_hbm,
                                    acc.at[pl.ds(s * 6400, 6400), :])
                    plsc.subcore_barrier()
                    issue(0, 0)
                    issue(1, 1)

                    def body(it, carry):
                        g = it * 2
                        wait_load(g, 0)
                        scatter(0)

                        @pl.when(g + 2 < NCH_H)
                        def _():
                            issue(g + 2, 0)

                        wait_load(g + 1, 1)
                        scatter(1)

                        @pl.when(g + 3 < NCH_H)
                        def _():
                            issue(g + 3, 1)

                        return carry

                    lax.fori_loop(0, NCH_H // 2, body, 0)
                    pltpu.sync_copy(
                        row_hbm.at[pl.ds(rbase + NCH_H * GC, GT)], idxtk)
                    pltpu.sync_copy(
                        h2_hbm.at[pl.ds(lbase + NCH_H * GC, GT),
                                  pl.ds(colbase, 16)],
                        rows0.at[pl.ds(0, GT), :])
                    pltpu.sync_copy(rows0.at[pl.ds(0, GT), :],
                                    acc.at[idxtk], add=True)
                    plsc.subcore_barrier()
                    pltpu.sync_copy(acc.at[pl.ds(s * 6400, 6400), :],
                                    sums_hbm.at[pl.ds(s * 6400, 6400),
                                                pl.ds(colbase, 16)])
                    plsc.subcore_barrier()

        if not with_counts:
            return

        pltpu.sync_copy(ones_hbm, rows0)
        for cc in range(NC):

            @pl.when(c == cc)
            def _(cc=cc):
                ebase = cc * (N_EDGES // NC) + s * EPT_K

                def issue_k(g, b):
                    pltpu.async_copy(row_hbm.at[pl.ds(ebase + g * GC, GC)],
                                     ibufs[b], isems[b])

                def wait_scatter_k(g, b):
                    pltpu.make_async_copy(
                        row_hbm.at[pl.ds(ebase + g * GC, GC)],
                        ibufs[b], isems[b]).wait()
                    pltpu.sync_copy(rows0, acc.at[ibufs[b]], add=True)

                pltpu.sync_copy(zeros2_hbm, acc.at[pl.ds(s * 6400, 6400), :])
                plsc.subcore_barrier()
                issue_k(0, 0)
                issue_k(1, 1)

                def body(it, carry):
                    g = it * 2
                    wait_scatter_k(g, 0)

                    @pl.when(g + 2 < NCH_K)
                    def _():
                        issue_k(g + 2, 0)

                    wait_scatter_k(g + 1, 1)

                    @pl.when(g + 3 < NCH_K)
                    def _():
                        issue_k(g + 3, 1)

                    return carry

                lax.fori_loop(0, NCH_K // 2, body, 0)
                pltpu.sync_copy(row_hbm.at[pl.ds(ebase + NCH_K * GC, GT)],
                                idxtk)
                pltpu.sync_copy(rows0.at[pl.ds(0, GT), :], acc.at[idxtk],
                                add=True)
                plsc.subcore_barrier()
                pltpu.sync_copy(acc.at[pl.ds(s * 6400, 6400), :],
                                klist[cc].at[pl.ds(s * 6400, 6400), :])
                plsc.subcore_barrier()

    return _phase_c


_phase_c_a = _make_phase_c(0, False)
_phase_c_b = _make_phase_c(1, True)


BN = 2000


def _phase_d_body(x, c0, c1, sa, sb, w3a, w3b, b3, w4, b4, out):
    ctot = c0[:, 0:1] + c1[:, 0:1]
    inv = 1.0 / jnp.maximum(ctot, 1.0)
    mean = (sa[...] + sb[...]) * inv
    h = jnp.dot(x[...], w3a[...], preferred_element_type=jnp.float32)
    h = h + jnp.dot(mean, w3b[...], preferred_element_type=jnp.float32)
    h = jnp.maximum(h + b3[...], 0.0)
    o = jnp.dot(h, w4[...], preferred_element_type=jnp.float32) + b4[...]
    out[...] = jnp.maximum(o, 0.0)


def _node_mlp(x, c0, c1, sa, sb, w3a, w3b, b3, w4, b4):
    grid = (N_NODES // BN,)
    return pl.pallas_call(
        _phase_d_body,
        grid=grid,
        in_specs=[
            pl.BlockSpec((BN, 4), lambda i: (i, 0)),
            pl.BlockSpec((BN, 16), lambda i: (i, 0)),
            pl.BlockSpec((BN, 16), lambda i: (i, 0)),
            pl.BlockSpec((BN, 128), lambda i: (i, 0)),
            pl.BlockSpec((BN, 128), lambda i: (i, 0)),
            pl.BlockSpec((4, 256), lambda i: (0, 0)),
            pl.BlockSpec((128, 256), lambda i: (0, 0)),
            pl.BlockSpec((1, 256), lambda i: (0, 0)),
            pl.BlockSpec((256, 256), lambda i: (0, 0)),
            pl.BlockSpec((1, 256), lambda i: (0, 0)),
        ],
        out_specs=pl.BlockSpec((BN, 256), lambda i: (i, 0)),
        out_shape=jax.ShapeDtypeStruct((N_NODES, 256), jnp.float32),
    )(x, c0, c1, sa, sb, w3a, w3b, b3, w4, b4)


def kernel(x, edge_index, edge_attr, W1, b1, W2, b2, W3, b3, W4, b4):
    row = edge_index[0]
    col = edge_index[1]
    x_pad = jnp.pad(x, ((0, 0), (0, 12)))
    w1a_pad = jnp.pad(W1[:4], ((0, 12), (0, 0)))
    ea_t = edge_attr.T
    zeros2 = jnp.zeros((6400, 16), jnp.float32)
    ones_c = jnp.ones((GC, 16), jnp.float32)
    b1r = b1.reshape(1, 64)
    b2r = b2.reshape(1, 128)

    xg = _phase_a(x_pad, col)
    h2a = _edge_mlp(0, xg, ea_t, w1a_pad, W1[4:], b1r, W2, b2r)
    h2b = _edge_mlp(1, xg, ea_t, w1a_pad, W1[4:], b1r, W2, b2r)
    (sums_a,) = _phase_c_a(h2a, row, zeros2, ones_c)
    sums_b, cnt0, cnt1 = _phase_c_b(h2b, row, zeros2, ones_c)
    out = _node_mlp(x, cnt0, cnt1, sums_a, sums_b, W3[:4], W3[4:],
                    b3.reshape(1, 256), W4, b4.reshape(1, 256))
    return out
